# Initial kernel scaffold; baseline (speedup 1.0000x reference)
import jax
import jax.numpy as jnp
from jax.experimental import pallas as pl


def kernel(h, x, edge_index, params):
    raise NotImplementedError("write your pallas kernel here")


# speedup vs baseline: 2.4008x; 2.4008x over previous
import functools

import jax
import jax.numpy as jnp
from jax import lax
from jax.experimental import pallas as pl
from jax.experimental.pallas import tpu as pltpu
from jax.experimental.pallas import tpu_sc as plsc

N = 10000
E = 320000
H = 64
CP = 8
NC = 2
NS = 16
NW = NC * NS
W = 128
CHUNK_WIN = 4
CHUNK = CHUNK_WIN * W
E_PAD = 327680
WIN_TOT = E_PAD // W
WIN_PER_WORKER = WIN_TOT // NW
NCHUNK = WIN_PER_WORKER // CHUNK_WIN
ROWS_PER_TILE = N // NS

BE = 2048
BN = 2000

_MESH = plsc.VectorSubcoreMesh(core_axis_name="c", subcore_axis_name="s")
_SC_PARAMS = pltpu.CompilerParams(use_tc_tiling_on_sc=False)
_f32 = jnp.float32


def _sc_gather(hs, ht, cp, row2d, col2d):
    @functools.partial(
        pl.kernel,
        out_type=[
            jax.ShapeDtypeStruct((E_PAD, H), _f32),
            jax.ShapeDtypeStruct((E_PAD, H), _f32),
            jax.ShapeDtypeStruct((E_PAD, CP), _f32),
            jax.ShapeDtypeStruct((E_PAD, CP), _f32),
        ],
        mesh=_MESH,
        scratch_types=[
            pltpu.VMEM((CHUNK_WIN, W), jnp.int32),
            pltpu.VMEM((CHUNK_WIN, W), jnp.int32),
            pltpu.VMEM((CHUNK, H), _f32),
            pltpu.VMEM((CHUNK, H), _f32),
            pltpu.VMEM((CHUNK, CP), _f32),
            pltpu.VMEM((CHUNK, CP), _f32),
            pltpu.SemaphoreType.DMA,
        ],
        compiler_params=_SC_PARAMS,
    )
    def k(hs_hbm, ht_hbm, cp_hbm, row_hbm, col_hbm,
          hsr_o, htr_o, cpr_o, cpc_o,
          idxr, idxc, hsr_v, htr_v, cpr_v, cpc_v, sem):
        wid = lax.axis_index("s") * NC + lax.axis_index("c")

        @pl.loop(0, NCHUNK)
        def _(ci):
            wbase = wid * WIN_PER_WORKER + ci * CHUNK_WIN
            ebase = wbase * W
            pltpu.sync_copy(row_hbm.at[pl.ds(wbase, CHUNK_WIN)], idxr)
            pltpu.sync_copy(col_hbm.at[pl.ds(wbase, CHUNK_WIN)], idxc)
            copies = []
            for j in range(CHUNK_WIN):
                sl = pl.ds(j * W, W)
                copies.append(pltpu.async_copy(hs_hbm.at[idxr.at[j]], hsr_v.at[sl], sem))
                copies.append(pltpu.async_copy(ht_hbm.at[idxc.at[j]], htr_v.at[sl], sem))
                copies.append(pltpu.async_copy(cp_hbm.at[idxr.at[j]], cpr_v.at[sl], sem))
                copies.append(pltpu.async_copy(cp_hbm.at[idxc.at[j]], cpc_v.at[sl], sem))
            for c in copies:
                c.wait()
            pltpu.sync_copy(hsr_v, hsr_o.at[pl.ds(ebase, CHUNK)])
            pltpu.sync_copy(htr_v, htr_o.at[pl.ds(ebase, CHUNK)])
            pltpu.sync_copy(cpr_v, cpr_o.at[pl.ds(ebase, CHUNK)])
            pltpu.sync_copy(cpc_v, cpc_o.at[pl.ds(ebase, CHUNK)])

    return k(hs, ht, cp, row2d, col2d)


def _sc_scatter(m, t8, row2d, zm, zx):
    @functools.partial(
        pl.kernel,
        out_type=[
            jax.ShapeDtypeStruct((NC, N, H), _f32),
            jax.ShapeDtypeStruct((NC, N, CP), _f32),
        ],
        mesh=_MESH,
        scratch_types=[
            pltpu.VMEM((CHUNK_WIN, W), jnp.int32),
            pltpu.VMEM((CHUNK, H), _f32),
            pltpu.VMEM((CHUNK, CP), _f32),
            pltpu.VMEM_SHARED((N, H), _f32),
            pltpu.VMEM_SHARED((N, CP), _f32),
        ],
        compiler_params=_SC_PARAMS,
    )
    def k(m_hbm, t8_hbm, row_hbm, zm_hbm, zx_hbm,
          mo, xo, idxr, m_v, t8_v, accm, accx):
        cid = lax.axis_index("c")
        sid = lax.axis_index("s")
        wid = sid * NC + cid
        rbase = sid * ROWS_PER_TILE
        rows = pl.ds(rbase, ROWS_PER_TILE)
        pltpu.sync_copy(zm_hbm, accm.at[rows])
        pltpu.sync_copy(zx_hbm, accx.at[rows])
        plsc.subcore_barrier()

        @pl.loop(0, NCHUNK)
        def _(ci):
            wbase = wid * WIN_PER_WORKER + ci * CHUNK_WIN
            ebase = wbase * W
            pltpu.sync_copy(row_hbm.at[pl.ds(wbase, CHUNK_WIN)], idxr)
            pltpu.sync_copy(m_hbm.at[pl.ds(ebase, CHUNK)], m_v)
            pltpu.sync_copy(t8_hbm.at[pl.ds(ebase, CHUNK)], t8_v)
            for j in range(CHUNK_WIN):
                sl = pl.ds(j * W, W)
                pltpu.sync_copy(m_v.at[sl], accm.at[idxr.at[j]], add=True)
                pltpu.sync_copy(t8_v.at[sl], accx.at[idxr.at[j]], add=True)

        plsc.subcore_barrier()
        pltpu.sync_copy(accm.at[rows], mo.at[cid, rows])
        pltpu.sync_copy(accx.at[rows], xo.at[cid, rows])

    return k(m, t8, row2d, zm, zx)


def _bf(a):
    return a.astype(jnp.bfloat16).astype(jnp.float32)


def _mm(a, b):
    return jnp.dot(a.astype(jnp.bfloat16), b.astype(jnp.bfloat16),
                   preferred_element_type=jnp.float32)


def _silu(x):
    return x * (0.5 + 0.5 * jnp.tanh(0.5 * x))


def _tc_edge(hsr, htr, cpr, cpc, w1r, b1e, w2, b2, cw1, cb1, cw2r):
    def body(hsr_r, htr_r, cpr_r, cpc_r,
             w1r_r, b1e_r, w2_r, b2_r, cw1_r, cb1_r, cw2_r, m_o, t8_o):
        i = pl.program_id(0)
        cd = cpr_r[...] - cpc_r[...]
        radial = jnp.sum(cd * cd, axis=1, keepdims=True)
        pre = hsr_r[...] + htr_r[...] + _bf(radial) * _bf(w1r_r[...]) + b1e_r[...]
        m1 = _silu(pre)
        m = _silu(_mm(m1, w2_r[...]) + b2_r[...])
        cm = _silu(_mm(m, cw1_r[...]) + cb1_r[...])
        s = jnp.sum(_bf(cm) * _bf(cw2_r[...]), axis=1, keepdims=True)
        t8 = cd * s
        rowid = i * BE + lax.broadcasted_iota(jnp.int32, (BE, 1), 0)
        valid = rowid < E
        colid = lax.broadcasted_iota(jnp.int32, (1, CP), 1)
        t8 = jnp.where(valid & (colid == 3), 1.0, t8)
        m_o[...] = jnp.where(valid, m, 0.0)
        t8_o[...] = t8

    full = lambda a: pl.BlockSpec(a.shape, lambda i: (0,) * a.ndim)
    return pl.pallas_call(
        body,
        grid=(E_PAD // BE,),
        in_specs=[
            pl.BlockSpec((BE, H), lambda i: (i, 0)),
            pl.BlockSpec((BE, H), lambda i: (i, 0)),
            pl.BlockSpec((BE, CP), lambda i: (i, 0)),
            pl.BlockSpec((BE, CP), lambda i: (i, 0)),
            full(w1r), full(b1e), full(w2), full(b2),
            full(cw1), full(cb1), full(cw2r),
        ],
        out_specs=[
            pl.BlockSpec((BE, H), lambda i: (i, 0)),
            pl.BlockSpec((BE, CP), lambda i: (i, 0)),
        ],
        out_shape=[
            jax.ShapeDtypeStruct((E_PAD, H), _f32),
            jax.ShapeDtypeStruct((E_PAD, CP), _f32),
        ],
    )(hsr, htr, cpr, cpc, w1r, b1e, w2, b2, cw1, cb1, cw2r)


def _tc_node(h64, maggp, aggxp, cp, nw1h, nw1m, nb1, nw2, nb2, wa, wb):
    def body(h_r, mg0_r, mg1_r, ax0_r, ax1_r, cp_r,
             nw1h_r, nw1m_r, nb1_r, nw2_r, nb2_r, wa_r, wb_r,
             h_o, cp_o, hs_o, ht_o):
        magg = mg0_r[0] + mg1_r[0]
        aggx = ax0_r[0] + ax1_r[0]
        cnt = aggx[:, 3:4]
        upd = aggx / jnp.maximum(cnt, 1.0)
        colid = lax.broadcasted_iota(jnp.int32, (1, CP), 1)
        cp_o[...] = cp_r[...] + jnp.where(colid < 3, upd, 0.0)
        tmp = _silu(_mm(h_r[...], nw1h_r[...])
                          + _mm(magg, nw1m_r[...]) + nb1_r[...])
        hn = h_r[...] + _mm(tmp, nw2_r[...]) + nb2_r[...]
        h_o[...] = hn
        hs_o[...] = _mm(hn, wa_r[...])
        ht_o[...] = _mm(hn, wb_r[...])

    full = lambda a: pl.BlockSpec(a.shape, lambda i: (0,) * a.ndim)
    return pl.pallas_call(
        body,
        grid=(N // BN,),
        in_specs=[
            pl.BlockSpec((BN, H), lambda i: (i, 0)),
            pl.BlockSpec((1, BN, H), lambda i: (0, i, 0)),
            pl.BlockSpec((1, BN, H), lambda i: (1, i, 0)),
            pl.BlockSpec((1, BN, CP), lambda i: (0, i, 0)),
            pl.BlockSpec((1, BN, CP), lambda i: (1, i, 0)),
            pl.BlockSpec((BN, CP), lambda i: (i, 0)),
            full(nw1h), full(nw1m), full(nb1), full(nw2), full(nb2),
            full(wa), full(wb),
        ],
        out_specs=[
            pl.BlockSpec((BN, H), lambda i: (i, 0)),
            pl.BlockSpec((BN, CP), lambda i: (i, 0)),
            pl.BlockSpec((BN, H), lambda i: (i, 0)),
            pl.BlockSpec((BN, H), lambda i: (i, 0)),
        ],
        out_shape=[
            jax.ShapeDtypeStruct((N, H), _f32),
            jax.ShapeDtypeStruct((N, CP), _f32),
            jax.ShapeDtypeStruct((N, H), _f32),
            jax.ShapeDtypeStruct((N, H), _f32),
        ],
    )(h64, maggp, maggp, aggxp, aggxp, cp,
      nw1h, nw1m, nb1, nw2, nb2, wa, wb)


def _tc_node_final(h64, maggp, aggxp, nw1h, nw1m, nb1, nw2, nb2,
                   ew, eb, mw1, mb1, mw2, mb2):
    nout = mw2.shape[1]

    def body(h_r, mg0_r, mg1_r,
             nw1h_r, nw1m_r, nb1_r, nw2_r, nb2_r,
             ew_r, eb_r, mw1_r, mb1_r, mw2_r, mb2_r, y_o):
        magg = mg0_r[0] + mg1_r[0]
        tmp = _silu(_mm(h_r[...], nw1h_r[...])
                          + _mm(magg, nw1m_r[...]) + nb1_r[...])
        hn = h_r[...] + _mm(tmp, nw2_r[...]) + nb2_r[...]
        t = _mm(hn, ew_r[...]) + eb_r[...]
        t2 = jnp.maximum(_mm(t, mw1_r[...]) + mb1_r[...], 0.0)
        y_o[...] = _mm(t2, mw2_r[...]) + mb2_r[...]

    full = lambda a: pl.BlockSpec(a.shape, lambda i: (0,) * a.ndim)
    return pl.pallas_call(
        body,
        grid=(N // BN,),
        in_specs=[
            pl.BlockSpec((BN, H), lambda i: (i, 0)),
            pl.BlockSpec((1, BN, H), lambda i: (0, i, 0)),
            pl.BlockSpec((1, BN, H), lambda i: (1, i, 0)),
            full(nw1h), full(nw1m), full(nb1), full(nw2), full(nb2),
            full(ew), full(eb), full(mw1), full(mb1), full(mw2), full(mb2),
        ],
        out_specs=pl.BlockSpec((BN, nout), lambda i: (i, 0)),
        out_shape=jax.ShapeDtypeStruct((N, nout), _f32),
    )(h64, maggp, maggp, nw1h, nw1m, nb1, nw2, nb2,
      ew, eb, mw1, mb1, mw2, mb2)


def _tc_init(h_in, ew, eb, wa, wb):
    def body(h_r, ew_r, eb_r, wa_r, wb_r, h_o, hs_o, ht_o):
        h64 = _mm(h_r[...], ew_r[...]) + eb_r[...]
        h_o[...] = h64
        hs_o[...] = _mm(h64, wa_r[...])
        ht_o[...] = _mm(h64, wb_r[...])

    full = lambda a: pl.BlockSpec(a.shape, lambda i: (0,) * a.ndim)
    din = h_in.shape[1]
    return pl.pallas_call(
        body,
        grid=(N // BN,),
        in_specs=[
            pl.BlockSpec((BN, din), lambda i: (i, 0)),
            full(ew), full(eb), full(wa), full(wb),
        ],
        out_specs=[
            pl.BlockSpec((BN, H), lambda i: (i, 0)),
            pl.BlockSpec((BN, H), lambda i: (i, 0)),
            pl.BlockSpec((BN, H), lambda i: (i, 0)),
        ],
        out_shape=[
            jax.ShapeDtypeStruct((N, H), _f32),
            jax.ShapeDtypeStruct((N, H), _f32),
            jax.ShapeDtypeStruct((N, H), _f32),
        ],
    )(h_in, ew, eb, wa, wb)


def kernel(h, x, edge_index, params):
    L = len(params["layers"])

    row = edge_index[:, 0]
    col = edge_index[:, 1]
    padn = E_PAD - E
    row2d = jnp.concatenate([row, jnp.zeros((padn,), jnp.int32)]).reshape(WIN_TOT, W)
    col2d = jnp.concatenate([col, jnp.zeros((padn,), jnp.int32)]).reshape(WIN_TOT, W)
    cp = jnp.pad(x, ((0, 0), (0, CP - x.shape[1])))
    zm = jnp.zeros((ROWS_PER_TILE, H), _f32)
    zx = jnp.zeros((ROWS_PER_TILE, CP), _f32)

    def r2(v):
        return v.reshape(1, -1)

    lw = []
    for p in params["layers"]:
        w1 = p["edge_w1"]
        lw.append(dict(
            wa=w1[:H], wb=w1[H:2 * H], w1r=r2(w1[2 * H]),
            b1e=r2(w1[2 * H + 1] + p["edge_b1"]),
            w2=p["edge_w2"], b2=r2(p["edge_b2"]),
            cw1=p["coord_w1"], cb1=r2(p["coord_b1"]), cw2r=r2(p["coord_w2"][:, 0]),
            nw1h=p["node_w1"][:H], nw1m=p["node_w1"][H:], nb1=r2(p["node_b1"]),
            nw2=p["node_w2"], nb2=r2(p["node_b2"]),
        ))

    h64, hs, ht = _tc_init(h, params["emb_in_w"], r2(params["emb_in_b"]),
                           lw[0]["wa"], lw[0]["wb"])

    y = None
    for l in range(L):
        p = lw[l]
        hsr, htr, cpr, cpc = _sc_gather(hs, ht, cp, row2d, col2d)
        m, t8 = _tc_edge(hsr, htr, cpr, cpc, p["w1r"], p["b1e"],
                         p["w2"], p["b2"], p["cw1"], p["cb1"], p["cw2r"])
        maggp, aggxp = _sc_scatter(m, t8, row2d, zm, zx)
        if l + 1 < L:
            h64, cp, hs, ht = _tc_node(
                h64, maggp, aggxp, cp,
                p["nw1h"], p["nw1m"], p["nb1"], p["nw2"], p["nb2"],
                lw[l + 1]["wa"], lw[l + 1]["wb"])
        else:
            y = _tc_node_final(
                h64, maggp, aggxp,
                p["nw1h"], p["nw1m"], p["nb1"], p["nw2"], p["nb2"],
                params["emb_out_w"], r2(params["emb_out_b"]),
                params["mlp_w1"], r2(params["mlp_b1"]),
                params["mlp_w2"], r2(params["mlp_b2"]))
    return y


# speedup vs baseline: 2.4352x; 1.0143x over previous
import functools

import jax
import jax.numpy as jnp
from jax import lax
from jax.experimental import pallas as pl
from jax.experimental.pallas import tpu as pltpu
from jax.experimental.pallas import tpu_sc as plsc

N = 10000
E = 320000
H = 64
CP = 8
NC = 2
NS = 16
NW = NC * NS
W = 128
CHUNK_WIN = 4
CHUNK = CHUNK_WIN * W
E_PAD = 327680
WIN_TOT = E_PAD // W
WIN_PER_WORKER = WIN_TOT // NW
NCHUNK = WIN_PER_WORKER // CHUNK_WIN
ROWS_PER_TILE = N // NS

BE = 8192
BN = 2000

_MESH = plsc.VectorSubcoreMesh(core_axis_name="c", subcore_axis_name="s")
_SC_PARAMS = pltpu.CompilerParams(use_tc_tiling_on_sc=False)
_f32 = jnp.float32


def _sc_gather(hs, ht, cp, row2d, col2d):
    @functools.partial(
        pl.kernel,
        out_type=[
            jax.ShapeDtypeStruct((E_PAD, H), _f32),
            jax.ShapeDtypeStruct((E_PAD, H), _f32),
            jax.ShapeDtypeStruct((E_PAD, CP), _f32),
            jax.ShapeDtypeStruct((E_PAD, CP), _f32),
        ],
        mesh=_MESH,
        scratch_types=[
            pltpu.VMEM((CHUNK_WIN, W), jnp.int32),
            pltpu.VMEM((CHUNK_WIN, W), jnp.int32),
            pltpu.VMEM((CHUNK, H), _f32),
            pltpu.VMEM((CHUNK, H), _f32),
            pltpu.VMEM((CHUNK, CP), _f32),
            pltpu.VMEM((CHUNK, CP), _f32),
            pltpu.SemaphoreType.DMA,
        ],
        compiler_params=_SC_PARAMS,
    )
    def k(hs_hbm, ht_hbm, cp_hbm, row_hbm, col_hbm,
          hsr_o, htr_o, cpr_o, cpc_o,
          idxr, idxc, hsr_v, htr_v, cpr_v, cpc_v, sem):
        wid = lax.axis_index("s") * NC + lax.axis_index("c")

        @pl.loop(0, NCHUNK)
        def _(ci):
            wbase = wid * WIN_PER_WORKER + ci * CHUNK_WIN
            ebase = wbase * W
            pltpu.sync_copy(row_hbm.at[pl.ds(wbase, CHUNK_WIN)], idxr)
            pltpu.sync_copy(col_hbm.at[pl.ds(wbase, CHUNK_WIN)], idxc)
            copies = []
            for j in range(CHUNK_WIN):
                sl = pl.ds(j * W, W)
                copies.append(pltpu.async_copy(hs_hbm.at[idxr.at[j]], hsr_v.at[sl], sem))
                copies.append(pltpu.async_copy(ht_hbm.at[idxc.at[j]], htr_v.at[sl], sem))
                copies.append(pltpu.async_copy(cp_hbm.at[idxr.at[j]], cpr_v.at[sl], sem))
                copies.append(pltpu.async_copy(cp_hbm.at[idxc.at[j]], cpc_v.at[sl], sem))
            for c in copies:
                c.wait()
            pltpu.sync_copy(hsr_v, hsr_o.at[pl.ds(ebase, CHUNK)])
            pltpu.sync_copy(htr_v, htr_o.at[pl.ds(ebase, CHUNK)])
            pltpu.sync_copy(cpr_v, cpr_o.at[pl.ds(ebase, CHUNK)])
            pltpu.sync_copy(cpc_v, cpc_o.at[pl.ds(ebase, CHUNK)])

    return k(hs, ht, cp, row2d, col2d)


def _sc_scatter(m, t8, row2d, zm, zx):
    @functools.partial(
        pl.kernel,
        out_type=[
            jax.ShapeDtypeStruct((NC, N, H), _f32),
            jax.ShapeDtypeStruct((NC, N, CP), _f32),
        ],
        mesh=_MESH,
        scratch_types=[
            pltpu.VMEM((CHUNK_WIN, W), jnp.int32),
            pltpu.VMEM((CHUNK, H), _f32),
            pltpu.VMEM((CHUNK, CP), _f32),
            pltpu.VMEM_SHARED((N, H), _f32),
            pltpu.VMEM_SHARED((N, CP), _f32),
        ],
        compiler_params=_SC_PARAMS,
    )
    def k(m_hbm, t8_hbm, row_hbm, zm_hbm, zx_hbm,
          mo, xo, idxr, m_v, t8_v, accm, accx):
        cid = lax.axis_index("c")
        sid = lax.axis_index("s")
        wid = sid * NC + cid
        rbase = sid * ROWS_PER_TILE
        rows = pl.ds(rbase, ROWS_PER_TILE)
        pltpu.sync_copy(zm_hbm, accm.at[rows])
        pltpu.sync_copy(zx_hbm, accx.at[rows])
        plsc.subcore_barrier()

        @pl.loop(0, NCHUNK)
        def _(ci):
            wbase = wid * WIN_PER_WORKER + ci * CHUNK_WIN
            ebase = wbase * W
            pltpu.sync_copy(row_hbm.at[pl.ds(wbase, CHUNK_WIN)], idxr)
            pltpu.sync_copy(m_hbm.at[pl.ds(ebase, CHUNK)], m_v)
            pltpu.sync_copy(t8_hbm.at[pl.ds(ebase, CHUNK)], t8_v)
            for j in range(CHUNK_WIN):
                sl = pl.ds(j * W, W)
                pltpu.sync_copy(m_v.at[sl], accm.at[idxr.at[j]], add=True)
                pltpu.sync_copy(t8_v.at[sl], accx.at[idxr.at[j]], add=True)

        plsc.subcore_barrier()
        pltpu.sync_copy(accm.at[rows], mo.at[cid, rows])
        pltpu.sync_copy(accx.at[rows], xo.at[cid, rows])

    return k(m, t8, row2d, zm, zx)


def _bf(a):
    return a.astype(jnp.bfloat16).astype(jnp.float32)


def _mm(a, b):
    return jnp.dot(a.astype(jnp.bfloat16), b.astype(jnp.bfloat16),
                   preferred_element_type=jnp.float32)


def _silu(x):
    return x * (0.5 + 0.5 * jnp.tanh(0.5 * x))


def _tc_edge(hsr, htr, cpr, cpc, w1r, b1e, w2, b2, cw1, cb1, cw2r):
    def body(hsr_r, htr_r, cpr_r, cpc_r,
             w1r_r, b1e_r, w2_r, b2_r, cw1_r, cb1_r, cw2_r, m_o, t8_o):
        i = pl.program_id(0)
        cd = cpr_r[...] - cpc_r[...]
        radial = jnp.sum(cd * cd, axis=1, keepdims=True)
        pre = hsr_r[...] + htr_r[...] + _bf(radial) * _bf(w1r_r[...]) + b1e_r[...]
        m1 = _silu(pre)
        m = _silu(_mm(m1, w2_r[...]) + b2_r[...])
        cm = _silu(_mm(m, cw1_r[...]) + cb1_r[...])
        s = jnp.sum(_bf(cm) * _bf(cw2_r[...]), axis=1, keepdims=True)
        t8 = cd * s
        rowid = i * BE + lax.broadcasted_iota(jnp.int32, (BE, 1), 0)
        valid = rowid < E
        colid = lax.broadcasted_iota(jnp.int32, (1, CP), 1)
        t8 = jnp.where(valid & (colid == 3), 1.0, t8)
        m_o[...] = jnp.where(valid, m, 0.0)
        t8_o[...] = t8

    full = lambda a: pl.BlockSpec(a.shape, lambda i: (0,) * a.ndim)
    return pl.pallas_call(
        body,
        grid=(E_PAD // BE,),
        in_specs=[
            pl.BlockSpec((BE, H), lambda i: (i, 0)),
            pl.BlockSpec((BE, H), lambda i: (i, 0)),
            pl.BlockSpec((BE, CP), lambda i: (i, 0)),
            pl.BlockSpec((BE, CP), lambda i: (i, 0)),
            full(w1r), full(b1e), full(w2), full(b2),
            full(cw1), full(cb1), full(cw2r),
        ],
        out_specs=[
            pl.BlockSpec((BE, H), lambda i: (i, 0)),
            pl.BlockSpec((BE, CP), lambda i: (i, 0)),
        ],
        out_shape=[
            jax.ShapeDtypeStruct((E_PAD, H), _f32),
            jax.ShapeDtypeStruct((E_PAD, CP), _f32),
        ],
    )(hsr, htr, cpr, cpc, w1r, b1e, w2, b2, cw1, cb1, cw2r)


def _tc_node(h64, maggp, aggxp, cp, nw1h, nw1m, nb1, nw2, nb2, wa, wb):
    def body(h_r, mg0_r, mg1_r, ax0_r, ax1_r, cp_r,
             nw1h_r, nw1m_r, nb1_r, nw2_r, nb2_r, wa_r, wb_r,
             h_o, cp_o, hs_o, ht_o):
        magg = mg0_r[0] + mg1_r[0]
        aggx = ax0_r[0] + ax1_r[0]
        cnt = aggx[:, 3:4]
        upd = aggx / jnp.maximum(cnt, 1.0)
        colid = lax.broadcasted_iota(jnp.int32, (1, CP), 1)
        cp_o[...] = cp_r[...] + jnp.where(colid < 3, upd, 0.0)
        tmp = _silu(_mm(h_r[...], nw1h_r[...])
                          + _mm(magg, nw1m_r[...]) + nb1_r[...])
        hn = h_r[...] + _mm(tmp, nw2_r[...]) + nb2_r[...]
        h_o[...] = hn
        hs_o[...] = _mm(hn, wa_r[...])
        ht_o[...] = _mm(hn, wb_r[...])

    full = lambda a: pl.BlockSpec(a.shape, lambda i: (0,) * a.ndim)
    return pl.pallas_call(
        body,
        grid=(N // BN,),
        in_specs=[
            pl.BlockSpec((BN, H), lambda i: (i, 0)),
            pl.BlockSpec((1, BN, H), lambda i: (0, i, 0)),
            pl.BlockSpec((1, BN, H), lambda i: (1, i, 0)),
            pl.BlockSpec((1, BN, CP), lambda i: (0, i, 0)),
            pl.BlockSpec((1, BN, CP), lambda i: (1, i, 0)),
            pl.BlockSpec((BN, CP), lambda i: (i, 0)),
            full(nw1h), full(nw1m), full(nb1), full(nw2), full(nb2),
            full(wa), full(wb),
        ],
        out_specs=[
            pl.BlockSpec((BN, H), lambda i: (i, 0)),
            pl.BlockSpec((BN, CP), lambda i: (i, 0)),
            pl.BlockSpec((BN, H), lambda i: (i, 0)),
            pl.BlockSpec((BN, H), lambda i: (i, 0)),
        ],
        out_shape=[
            jax.ShapeDtypeStruct((N, H), _f32),
            jax.ShapeDtypeStruct((N, CP), _f32),
            jax.ShapeDtypeStruct((N, H), _f32),
            jax.ShapeDtypeStruct((N, H), _f32),
        ],
    )(h64, maggp, maggp, aggxp, aggxp, cp,
      nw1h, nw1m, nb1, nw2, nb2, wa, wb)


def _tc_node_final(h64, maggp, aggxp, nw1h, nw1m, nb1, nw2, nb2,
                   ew, eb, mw1, mb1, mw2, mb2):
    nout = mw2.shape[1]

    def body(h_r, mg0_r, mg1_r,
             nw1h_r, nw1m_r, nb1_r, nw2_r, nb2_r,
             ew_r, eb_r, mw1_r, mb1_r, mw2_r, mb2_r, y_o):
        magg = mg0_r[0] + mg1_r[0]
        tmp = _silu(_mm(h_r[...], nw1h_r[...])
                          + _mm(magg, nw1m_r[...]) + nb1_r[...])
        hn = h_r[...] + _mm(tmp, nw2_r[...]) + nb2_r[...]
        t = _mm(hn, ew_r[...]) + eb_r[...]
        t2 = jnp.maximum(_mm(t, mw1_r[...]) + mb1_r[...], 0.0)
        y_o[...] = _mm(t2, mw2_r[...]) + mb2_r[...]

    full = lambda a: pl.BlockSpec(a.shape, lambda i: (0,) * a.ndim)
    return pl.pallas_call(
        body,
        grid=(N // BN,),
        in_specs=[
            pl.BlockSpec((BN, H), lambda i: (i, 0)),
            pl.BlockSpec((1, BN, H), lambda i: (0, i, 0)),
            pl.BlockSpec((1, BN, H), lambda i: (1, i, 0)),
            full(nw1h), full(nw1m), full(nb1), full(nw2), full(nb2),
            full(ew), full(eb), full(mw1), full(mb1), full(mw2), full(mb2),
        ],
        out_specs=pl.BlockSpec((BN, nout), lambda i: (i, 0)),
        out_shape=jax.ShapeDtypeStruct((N, nout), _f32),
    )(h64, maggp, maggp, nw1h, nw1m, nb1, nw2, nb2,
      ew, eb, mw1, mb1, mw2, mb2)


def _tc_init(h_in, ew, eb, wa, wb):
    def body(h_r, ew_r, eb_r, wa_r, wb_r, h_o, hs_o, ht_o):
        h64 = _mm(h_r[...], ew_r[...]) + eb_r[...]
        h_o[...] = h64
        hs_o[...] = _mm(h64, wa_r[...])
        ht_o[...] = _mm(h64, wb_r[...])

    full = lambda a: pl.BlockSpec(a.shape, lambda i: (0,) * a.ndim)
    din = h_in.shape[1]
    return pl.pallas_call(
        body,
        grid=(N // BN,),
        in_specs=[
            pl.BlockSpec((BN, din), lambda i: (i, 0)),
            full(ew), full(eb), full(wa), full(wb),
        ],
        out_specs=[
            pl.BlockSpec((BN, H), lambda i: (i, 0)),
            pl.BlockSpec((BN, H), lambda i: (i, 0)),
            pl.BlockSpec((BN, H), lambda i: (i, 0)),
        ],
        out_shape=[
            jax.ShapeDtypeStruct((N, H), _f32),
            jax.ShapeDtypeStruct((N, H), _f32),
            jax.ShapeDtypeStruct((N, H), _f32),
        ],
    )(h_in, ew, eb, wa, wb)


def kernel(h, x, edge_index, params):
    L = len(params["layers"])

    row = edge_index[:, 0]
    col = edge_index[:, 1]
    padn = E_PAD - E
    row2d = jnp.concatenate([row, jnp.zeros((padn,), jnp.int32)]).reshape(WIN_TOT, W)
    col2d = jnp.concatenate([col, jnp.zeros((padn,), jnp.int32)]).reshape(WIN_TOT, W)
    cp = jnp.pad(x, ((0, 0), (0, CP - x.shape[1])))
    zm = jnp.zeros((ROWS_PER_TILE, H), _f32)
    zx = jnp.zeros((ROWS_PER_TILE, CP), _f32)

    def r2(v):
        return v.reshape(1, -1)

    lw = []
    for p in params["layers"]:
        w1 = p["edge_w1"]
        lw.append(dict(
            wa=w1[:H], wb=w1[H:2 * H], w1r=r2(w1[2 * H]),
            b1e=r2(w1[2 * H + 1] + p["edge_b1"]),
            w2=p["edge_w2"], b2=r2(p["edge_b2"]),
            cw1=p["coord_w1"], cb1=r2(p["coord_b1"]), cw2r=r2(p["coord_w2"][:, 0]),
            nw1h=p["node_w1"][:H], nw1m=p["node_w1"][H:], nb1=r2(p["node_b1"]),
            nw2=p["node_w2"], nb2=r2(p["node_b2"]),
        ))

    h64, hs, ht = _tc_init(h, params["emb_in_w"], r2(params["emb_in_b"]),
                           lw[0]["wa"], lw[0]["wb"])

    y = None
    for l in range(L):
        p = lw[l]
        hsr, htr, cpr, cpc = _sc_gather(hs, ht, cp, row2d, col2d)
        m, t8 = _tc_edge(hsr, htr, cpr, cpc, p["w1r"], p["b1e"],
                         p["w2"], p["b2"], p["cw1"], p["cb1"], p["cw2r"])
        maggp, aggxp = _sc_scatter(m, t8, row2d, zm, zx)
        if l + 1 < L:
            h64, cp, hs, ht = _tc_node(
                h64, maggp, aggxp, cp,
                p["nw1h"], p["nw1m"], p["nb1"], p["nw2"], p["nb2"],
                lw[l + 1]["wa"], lw[l + 1]["wb"])
        else:
            y = _tc_node_final(
                h64, maggp, aggxp,
                p["nw1h"], p["nw1m"], p["nb1"], p["nw2"], p["nb2"],
                params["emb_out_w"], r2(params["emb_out_b"]),
                params["mlp_w1"], r2(params["mlp_b1"]),
                params["mlp_w2"], r2(params["mlp_b2"]))
    return y


# speedup vs baseline: 2.4601x; 1.0102x over previous
import functools

import jax
import jax.numpy as jnp
from jax import lax
from jax.experimental import pallas as pl
from jax.experimental.pallas import tpu as pltpu
from jax.experimental.pallas import tpu_sc as plsc

N = 10000
E = 320000
H = 64
CP = 8
NC = 2
NS = 16
NW = NC * NS
W = 128
CHUNK_WIN = 4
CHUNK = CHUNK_WIN * W
E_PAD = 327680
WIN_TOT = E_PAD // W
WIN_PER_WORKER = WIN_TOT // NW
NCHUNK = WIN_PER_WORKER // CHUNK_WIN
ROWS_PER_TILE = N // NS

BE = 8192
BN = 2000

_MESH = plsc.VectorSubcoreMesh(core_axis_name="c", subcore_axis_name="s")
_SC_PARAMS = pltpu.CompilerParams(use_tc_tiling_on_sc=False)
_f32 = jnp.float32


WIN_BLK = WIN_TOT // NS
WIN_SC0 = 96
WIN_SC1 = WIN_BLK - WIN_SC0
WPI = 4
SUB = 2 * W


def _sc_gather(hs, ht, cp, row2d, col2d):
    @functools.partial(
        pl.kernel,
        out_type=[
            jax.ShapeDtypeStruct((E_PAD, H), _f32),
            jax.ShapeDtypeStruct((E_PAD, H), _f32),
            jax.ShapeDtypeStruct((E_PAD, CP), _f32),
            jax.ShapeDtypeStruct((E_PAD, CP), _f32),
        ],
        mesh=_MESH,
        scratch_types=[
            pltpu.VMEM((WPI, W), jnp.int32),
            pltpu.VMEM((WPI, W), jnp.int32),
            [pltpu.VMEM((SUB, H), _f32)] * 2,
            [pltpu.VMEM((SUB, H), _f32)] * 2,
            [pltpu.VMEM((SUB, CP), _f32)] * 2,
            [pltpu.VMEM((SUB, CP), _f32)] * 2,
            pltpu.SemaphoreType.DMA,
            pltpu.SemaphoreType.DMA,
        ],
        compiler_params=_SC_PARAMS,
    )
    def k(hs_hbm, ht_hbm, cp_hbm, row_hbm, col_hbm,
          hsr_o, htr_o, cpr_o, cpc_o,
          idxr, idxc, hsr_v, htr_v, cpr_v, cpc_v, semg, sems):
        cid = lax.axis_index("c")
        sid = lax.axis_index("s")

        def do_iter(wb):
            eb = wb * W
            pltpu.sync_copy(row_hbm.at[pl.ds(wb, WPI)], idxr)
            pltpu.sync_copy(col_hbm.at[pl.ds(wb, WPI)], idxc)
            gat = [[], []]
            for s in range(2):
                for j in range(2):
                    wj = s * 2 + j
                    sl = pl.ds(j * W, W)
                    gat[s] += [
                        pltpu.async_copy(hs_hbm.at[idxr.at[wj]], hsr_v[s].at[sl], semg),
                        pltpu.async_copy(ht_hbm.at[idxc.at[wj]], htr_v[s].at[sl], semg),
                        pltpu.async_copy(cp_hbm.at[idxr.at[wj]], cpr_v[s].at[sl], semg),
                        pltpu.async_copy(cp_hbm.at[idxc.at[wj]], cpc_v[s].at[sl], semg),
                    ]
            st = []
            for s in range(2):
                for c in gat[s]:
                    c.wait()
                ebs = eb + s * SUB
                st += [
                    pltpu.async_copy(hsr_v[s], hsr_o.at[pl.ds(ebs, SUB)], sems),
                    pltpu.async_copy(htr_v[s], htr_o.at[pl.ds(ebs, SUB)], sems),
                    pltpu.async_copy(cpr_v[s], cpr_o.at[pl.ds(ebs, SUB)], sems),
                    pltpu.async_copy(cpc_v[s], cpc_o.at[pl.ds(ebs, SUB)], sems),
                ]
            for c in st:
                c.wait()

        @pl.when(cid == 0)
        def _():
            @pl.loop(0, WIN_SC0 // WPI)
            def _(ci):
                do_iter(sid * WIN_BLK + ci * WPI)

        @pl.when(cid == 1)
        def _():
            @pl.loop(0, WIN_SC1 // WPI)
            def _(ci):
                do_iter(sid * WIN_BLK + WIN_SC0 + ci * WPI)

    return k(hs, ht, cp, row2d, col2d)


def _sc_scatter(m, t8, row2d, zm, zx):
    @functools.partial(
        pl.kernel,
        out_type=[
            jax.ShapeDtypeStruct((NC, N, H), _f32),
            jax.ShapeDtypeStruct((NC, N, CP), _f32),
        ],
        mesh=_MESH,
        scratch_types=[
            pltpu.VMEM((CHUNK_WIN, W), jnp.int32),
            pltpu.VMEM((CHUNK, H), _f32),
            pltpu.VMEM((CHUNK, CP), _f32),
            pltpu.VMEM_SHARED((N, H), _f32),
            pltpu.VMEM_SHARED((N, CP), _f32),
        ],
        compiler_params=_SC_PARAMS,
    )
    def k(m_hbm, t8_hbm, row_hbm, zm_hbm, zx_hbm,
          mo, xo, idxr, m_v, t8_v, accm, accx):
        cid = lax.axis_index("c")
        sid = lax.axis_index("s")
        wid = sid * NC + cid
        rbase = sid * ROWS_PER_TILE
        rows = pl.ds(rbase, ROWS_PER_TILE)
        pltpu.sync_copy(zm_hbm, accm.at[rows])
        pltpu.sync_copy(zx_hbm, accx.at[rows])
        plsc.subcore_barrier()

        @pl.loop(0, NCHUNK)
        def _(ci):
            wbase = wid * WIN_PER_WORKER + ci * CHUNK_WIN
            ebase = wbase * W
            pltpu.sync_copy(row_hbm.at[pl.ds(wbase, CHUNK_WIN)], idxr)
            pltpu.sync_copy(m_hbm.at[pl.ds(ebase, CHUNK)], m_v)
            pltpu.sync_copy(t8_hbm.at[pl.ds(ebase, CHUNK)], t8_v)
            for j in range(CHUNK_WIN):
                sl = pl.ds(j * W, W)
                pltpu.sync_copy(m_v.at[sl], accm.at[idxr.at[j]], add=True)
                pltpu.sync_copy(t8_v.at[sl], accx.at[idxr.at[j]], add=True)

        plsc.subcore_barrier()
        pltpu.sync_copy(accm.at[rows], mo.at[cid, rows])
        pltpu.sync_copy(accx.at[rows], xo.at[cid, rows])

    return k(m, t8, row2d, zm, zx)


def _bf(a):
    return a.astype(jnp.bfloat16).astype(jnp.float32)


def _mm(a, b):
    return jnp.dot(a.astype(jnp.bfloat16), b.astype(jnp.bfloat16),
                   preferred_element_type=jnp.float32)


def _silu(x):
    return x * (0.5 + 0.5 * jnp.tanh(0.5 * x))


def _tc_edge(hsr, htr, cpr, cpc, w1r, b1e, w2, b2, cw1, cb1, cw2r):
    def body(hsr_r, htr_r, cpr_r, cpc_r,
             w1r_r, b1e_r, w2_r, b2_r, cw1_r, cb1_r, cw2_r, m_o, t8_o):
        i = pl.program_id(0)
        cd = cpr_r[...] - cpc_r[...]
        radial = jnp.sum(cd * cd, axis=1, keepdims=True)
        pre = hsr_r[...] + htr_r[...] + _bf(radial) * _bf(w1r_r[...]) + b1e_r[...]
        m1 = _silu(pre)
        m = _silu(_mm(m1, w2_r[...]) + b2_r[...])
        cm = _silu(_mm(m, cw1_r[...]) + cb1_r[...])
        s = jnp.sum(_bf(cm) * _bf(cw2_r[...]), axis=1, keepdims=True)
        t8 = cd * s
        rowid = i * BE + lax.broadcasted_iota(jnp.int32, (BE, 1), 0)
        valid = rowid < E
        colid = lax.broadcasted_iota(jnp.int32, (1, CP), 1)
        t8 = jnp.where(valid & (colid == 3), 1.0, t8)
        m_o[...] = jnp.where(valid, m, 0.0)
        t8_o[...] = t8

    full = lambda a: pl.BlockSpec(a.shape, lambda i: (0,) * a.ndim)
    return pl.pallas_call(
        body,
        grid=(E_PAD // BE,),
        in_specs=[
            pl.BlockSpec((BE, H), lambda i: (i, 0)),
            pl.BlockSpec((BE, H), lambda i: (i, 0)),
            pl.BlockSpec((BE, CP), lambda i: (i, 0)),
            pl.BlockSpec((BE, CP), lambda i: (i, 0)),
            full(w1r), full(b1e), full(w2), full(b2),
            full(cw1), full(cb1), full(cw2r),
        ],
        out_specs=[
            pl.BlockSpec((BE, H), lambda i: (i, 0)),
            pl.BlockSpec((BE, CP), lambda i: (i, 0)),
        ],
        out_shape=[
            jax.ShapeDtypeStruct((E_PAD, H), _f32),
            jax.ShapeDtypeStruct((E_PAD, CP), _f32),
        ],
    )(hsr, htr, cpr, cpc, w1r, b1e, w2, b2, cw1, cb1, cw2r)


def _tc_node(h64, maggp, aggxp, cp, nw1h, nw1m, nb1, nw2, nb2, wa, wb):
    def body(h_r, mg0_r, mg1_r, ax0_r, ax1_r, cp_r,
             nw1h_r, nw1m_r, nb1_r, nw2_r, nb2_r, wa_r, wb_r,
             h_o, cp_o, hs_o, ht_o):
        magg = mg0_r[0] + mg1_r[0]
        aggx = ax0_r[0] + ax1_r[0]
        cnt = aggx[:, 3:4]
        upd = aggx / jnp.maximum(cnt, 1.0)
        colid = lax.broadcasted_iota(jnp.int32, (1, CP), 1)
        cp_o[...] = cp_r[...] + jnp.where(colid < 3, upd, 0.0)
        tmp = _silu(_mm(h_r[...], nw1h_r[...])
                          + _mm(magg, nw1m_r[...]) + nb1_r[...])
        hn = h_r[...] + _mm(tmp, nw2_r[...]) + nb2_r[...]
        h_o[...] = hn
        hs_o[...] = _mm(hn, wa_r[...])
        ht_o[...] = _mm(hn, wb_r[...])

    full = lambda a: pl.BlockSpec(a.shape, lambda i: (0,) * a.ndim)
    return pl.pallas_call(
        body,
        grid=(N // BN,),
        in_specs=[
            pl.BlockSpec((BN, H), lambda i: (i, 0)),
            pl.BlockSpec((1, BN, H), lambda i: (0, i, 0)),
            pl.BlockSpec((1, BN, H), lambda i: (1, i, 0)),
            pl.BlockSpec((1, BN, CP), lambda i: (0, i, 0)),
            pl.BlockSpec((1, BN, CP), lambda i: (1, i, 0)),
            pl.BlockSpec((BN, CP), lambda i: (i, 0)),
            full(nw1h), full(nw1m), full(nb1), full(nw2), full(nb2),
            full(wa), full(wb),
        ],
        out_specs=[
            pl.BlockSpec((BN, H), lambda i: (i, 0)),
            pl.BlockSpec((BN, CP), lambda i: (i, 0)),
            pl.BlockSpec((BN, H), lambda i: (i, 0)),
            pl.BlockSpec((BN, H), lambda i: (i, 0)),
        ],
        out_shape=[
            jax.ShapeDtypeStruct((N, H), _f32),
            jax.ShapeDtypeStruct((N, CP), _f32),
            jax.ShapeDtypeStruct((N, H), _f32),
            jax.ShapeDtypeStruct((N, H), _f32),
        ],
    )(h64, maggp, maggp, aggxp, aggxp, cp,
      nw1h, nw1m, nb1, nw2, nb2, wa, wb)


def _tc_node_final(h64, maggp, aggxp, nw1h, nw1m, nb1, nw2, nb2,
                   ew, eb, mw1, mb1, mw2, mb2):
    nout = mw2.shape[1]

    def body(h_r, mg0_r, mg1_r,
             nw1h_r, nw1m_r, nb1_r, nw2_r, nb2_r,
             ew_r, eb_r, mw1_r, mb1_r, mw2_r, mb2_r, y_o):
        magg = mg0_r[0] + mg1_r[0]
        tmp = _silu(_mm(h_r[...], nw1h_r[...])
                          + _mm(magg, nw1m_r[...]) + nb1_r[...])
        hn = h_r[...] + _mm(tmp, nw2_r[...]) + nb2_r[...]
        t = _mm(hn, ew_r[...]) + eb_r[...]
        t2 = jnp.maximum(_mm(t, mw1_r[...]) + mb1_r[...], 0.0)
        y_o[...] = _mm(t2, mw2_r[...]) + mb2_r[...]

    full = lambda a: pl.BlockSpec(a.shape, lambda i: (0,) * a.ndim)
    return pl.pallas_call(
        body,
        grid=(N // BN,),
        in_specs=[
            pl.BlockSpec((BN, H), lambda i: (i, 0)),
            pl.BlockSpec((1, BN, H), lambda i: (0, i, 0)),
            pl.BlockSpec((1, BN, H), lambda i: (1, i, 0)),
            full(nw1h), full(nw1m), full(nb1), full(nw2), full(nb2),
            full(ew), full(eb), full(mw1), full(mb1), full(mw2), full(mb2),
        ],
        out_specs=pl.BlockSpec((BN, nout), lambda i: (i, 0)),
        out_shape=jax.ShapeDtypeStruct((N, nout), _f32),
    )(h64, maggp, maggp, nw1h, nw1m, nb1, nw2, nb2,
      ew, eb, mw1, mb1, mw2, mb2)


def _tc_init(h_in, ew, eb, wa, wb):
    def body(h_r, ew_r, eb_r, wa_r, wb_r, h_o, hs_o, ht_o):
        h64 = _mm(h_r[...], ew_r[...]) + eb_r[...]
        h_o[...] = h64
        hs_o[...] = _mm(h64, wa_r[...])
        ht_o[...] = _mm(h64, wb_r[...])

    full = lambda a: pl.BlockSpec(a.shape, lambda i: (0,) * a.ndim)
    din = h_in.shape[1]
    return pl.pallas_call(
        body,
        grid=(N // BN,),
        in_specs=[
            pl.BlockSpec((BN, din), lambda i: (i, 0)),
            full(ew), full(eb), full(wa), full(wb),
        ],
        out_specs=[
            pl.BlockSpec((BN, H), lambda i: (i, 0)),
            pl.BlockSpec((BN, H), lambda i: (i, 0)),
            pl.BlockSpec((BN, H), lambda i: (i, 0)),
        ],
        out_shape=[
            jax.ShapeDtypeStruct((N, H), _f32),
            jax.ShapeDtypeStruct((N, H), _f32),
            jax.ShapeDtypeStruct((N, H), _f32),
        ],
    )(h_in, ew, eb, wa, wb)


def kernel(h, x, edge_index, params):
    L = len(params["layers"])

    row = edge_index[:, 0]
    col = edge_index[:, 1]
    padn = E_PAD - E
    row2d = jnp.concatenate([row, jnp.zeros((padn,), jnp.int32)]).reshape(WIN_TOT, W)
    col2d = jnp.concatenate([col, jnp.zeros((padn,), jnp.int32)]).reshape(WIN_TOT, W)
    cp = jnp.pad(x, ((0, 0), (0, CP - x.shape[1])))
    zm = jnp.zeros((ROWS_PER_TILE, H), _f32)
    zx = jnp.zeros((ROWS_PER_TILE, CP), _f32)

    def r2(v):
        return v.reshape(1, -1)

    lw = []
    for p in params["layers"]:
        w1 = p["edge_w1"]
        lw.append(dict(
            wa=w1[:H], wb=w1[H:2 * H], w1r=r2(w1[2 * H]),
            b1e=r2(w1[2 * H + 1] + p["edge_b1"]),
            w2=p["edge_w2"], b2=r2(p["edge_b2"]),
            cw1=p["coord_w1"], cb1=r2(p["coord_b1"]), cw2r=r2(p["coord_w2"][:, 0]),
            nw1h=p["node_w1"][:H], nw1m=p["node_w1"][H:], nb1=r2(p["node_b1"]),
            nw2=p["node_w2"], nb2=r2(p["node_b2"]),
        ))

    h64, hs, ht = _tc_init(h, params["emb_in_w"], r2(params["emb_in_b"]),
                           lw[0]["wa"], lw[0]["wb"])

    y = None
    for l in range(L):
        p = lw[l]
        hsr, htr, cpr, cpc = _sc_gather(hs, ht, cp, row2d, col2d)
        m, t8 = _tc_edge(hsr, htr, cpr, cpc, p["w1r"], p["b1e"],
                         p["w2"], p["b2"], p["cw1"], p["cb1"], p["cw2r"])
        maggp, aggxp = _sc_scatter(m, t8, row2d, zm, zx)
        if l + 1 < L:
            h64, cp, hs, ht = _tc_node(
                h64, maggp, aggxp, cp,
                p["nw1h"], p["nw1m"], p["nb1"], p["nw2"], p["nb2"],
                lw[l + 1]["wa"], lw[l + 1]["wb"])
        else:
            y = _tc_node_final(
                h64, maggp, aggxp,
                p["nw1h"], p["nw1m"], p["nb1"], p["nw2"], p["nb2"],
                params["emb_out_w"], r2(params["emb_out_b"]),
                params["mlp_w1"], r2(params["mlp_b1"]),
                params["mlp_w2"], r2(params["mlp_b2"]))
    return y


# speedup vs baseline: 2.5362x; 1.0309x over previous
import functools

import jax
import jax.numpy as jnp
from jax import lax
from jax.experimental import pallas as pl
from jax.experimental.pallas import tpu as pltpu
from jax.experimental.pallas import tpu_sc as plsc

N = 10000
E = 320000
H = 64
CP = 8
NC = 2
NS = 16
NW = NC * NS
W = 128
CHUNK_WIN = 4
CHUNK = CHUNK_WIN * W
E_PAD = 327680
WIN_TOT = E_PAD // W
WIN_PER_WORKER = WIN_TOT // NW
NCHUNK = WIN_PER_WORKER // CHUNK_WIN
ROWS_PER_TILE = N // NS

BE = 8192
BN = 2000

_MESH = plsc.VectorSubcoreMesh(core_axis_name="c", subcore_axis_name="s")
_SC_PARAMS = pltpu.CompilerParams(use_tc_tiling_on_sc=False)
_f32 = jnp.float32


WIN_BLK = WIN_TOT // NS
WIN_SC0 = 120
WIN_SC1 = WIN_BLK - WIN_SC0
WPI = 4
SUB = 2 * W


def _sc_gather(hs, ht, cp, row2d, col2d):
    @functools.partial(
        pl.kernel,
        out_type=[
            jax.ShapeDtypeStruct((E_PAD, H), _f32),
            jax.ShapeDtypeStruct((E_PAD, H), _f32),
            jax.ShapeDtypeStruct((E_PAD, CP), _f32),
            jax.ShapeDtypeStruct((E_PAD, CP), _f32),
        ],
        mesh=_MESH,
        scratch_types=[
            pltpu.VMEM((WPI, W), jnp.int32),
            pltpu.VMEM((WPI, W), jnp.int32),
            [pltpu.VMEM((SUB, H), _f32)] * 2,
            [pltpu.VMEM((SUB, H), _f32)] * 2,
            [pltpu.VMEM((SUB, CP), _f32)] * 2,
            [pltpu.VMEM((SUB, CP), _f32)] * 2,
            pltpu.SemaphoreType.DMA,
            pltpu.SemaphoreType.DMA,
        ],
        compiler_params=_SC_PARAMS,
    )
    def k(hs_hbm, ht_hbm, cp_hbm, row_hbm, col_hbm,
          hsr_o, htr_o, cpr_o, cpc_o,
          idxr, idxc, hsr_v, htr_v, cpr_v, cpc_v, semg, sems):
        cid = lax.axis_index("c")
        sid = lax.axis_index("s")

        def do_iter(wb):
            eb = wb * W
            pltpu.sync_copy(row_hbm.at[pl.ds(wb, WPI)], idxr)
            pltpu.sync_copy(col_hbm.at[pl.ds(wb, WPI)], idxc)
            gat = [[], []]
            for s in range(2):
                for j in range(2):
                    wj = s * 2 + j
                    sl = pl.ds(j * W, W)
                    gat[s] += [
                        pltpu.async_copy(hs_hbm.at[idxr.at[wj]], hsr_v[s].at[sl], semg),
                        pltpu.async_copy(ht_hbm.at[idxc.at[wj]], htr_v[s].at[sl], semg),
                        pltpu.async_copy(cp_hbm.at[idxr.at[wj]], cpr_v[s].at[sl], semg),
                        pltpu.async_copy(cp_hbm.at[idxc.at[wj]], cpc_v[s].at[sl], semg),
                    ]
            st = []
            for s in range(2):
                for c in gat[s]:
                    c.wait()
                ebs = eb + s * SUB
                st += [
                    pltpu.async_copy(hsr_v[s], hsr_o.at[pl.ds(ebs, SUB)], sems),
                    pltpu.async_copy(htr_v[s], htr_o.at[pl.ds(ebs, SUB)], sems),
                    pltpu.async_copy(cpr_v[s], cpr_o.at[pl.ds(ebs, SUB)], sems),
                    pltpu.async_copy(cpc_v[s], cpc_o.at[pl.ds(ebs, SUB)], sems),
                ]
            for c in st:
                c.wait()

        @pl.when(cid == 0)
        def _():
            @pl.loop(0, WIN_SC0 // WPI)
            def _(ci):
                do_iter(sid * WIN_BLK + ci * WPI)

        @pl.when(cid == 1)
        def _():
            @pl.loop(0, WIN_SC1 // WPI)
            def _(ci):
                do_iter(sid * WIN_BLK + WIN_SC0 + ci * WPI)

    return k(hs, ht, cp, row2d, col2d)


def _sc_scatter(m, t8, row2d, zm, zx):
    @functools.partial(
        pl.kernel,
        out_type=[
            jax.ShapeDtypeStruct((NC, N, H), _f32),
            jax.ShapeDtypeStruct((NC, N, CP), _f32),
        ],
        mesh=_MESH,
        scratch_types=[
            pltpu.VMEM((CHUNK_WIN, W), jnp.int32),
            pltpu.VMEM((CHUNK, H), _f32),
            pltpu.VMEM((CHUNK, CP), _f32),
            pltpu.VMEM_SHARED((N, H), _f32),
            pltpu.VMEM_SHARED((N, CP), _f32),
        ],
        compiler_params=_SC_PARAMS,
    )
    def k(m_hbm, t8_hbm, row_hbm, zm_hbm, zx_hbm,
          mo, xo, idxr, m_v, t8_v, accm, accx):
        cid = lax.axis_index("c")
        sid = lax.axis_index("s")
        wid = sid * NC + cid
        rbase = sid * ROWS_PER_TILE
        rows = pl.ds(rbase, ROWS_PER_TILE)
        pltpu.sync_copy(zm_hbm, accm.at[rows])
        pltpu.sync_copy(zx_hbm, accx.at[rows])
        plsc.subcore_barrier()

        @pl.loop(0, NCHUNK)
        def _(ci):
            wbase = wid * WIN_PER_WORKER + ci * CHUNK_WIN
            ebase = wbase * W
            pltpu.sync_copy(row_hbm.at[pl.ds(wbase, CHUNK_WIN)], idxr)
            pltpu.sync_copy(m_hbm.at[pl.ds(ebase, CHUNK)], m_v)
            pltpu.sync_copy(t8_hbm.at[pl.ds(ebase, CHUNK)], t8_v)
            for j in range(CHUNK_WIN):
                sl = pl.ds(j * W, W)
                pltpu.sync_copy(m_v.at[sl], accm.at[idxr.at[j]], add=True)
                pltpu.sync_copy(t8_v.at[sl], accx.at[idxr.at[j]], add=True)

        plsc.subcore_barrier()
        pltpu.sync_copy(accm.at[rows], mo.at[cid, rows])
        pltpu.sync_copy(accx.at[rows], xo.at[cid, rows])

    return k(m, t8, row2d, zm, zx)


def _bf(a):
    return a.astype(jnp.bfloat16).astype(jnp.float32)


def _mm(a, b):
    return jnp.dot(a.astype(jnp.bfloat16), b.astype(jnp.bfloat16),
                   preferred_element_type=jnp.float32)


def _silu(x):
    return x * (0.5 + 0.5 * jnp.tanh(0.5 * x))


def _tc_edge(hsr, htr, cpr, cpc, w1r, b1e, w2, b2, cw1, cb1, cw2r):
    def body(hsr_r, htr_r, cpr_r, cpc_r,
             w1r_r, b1e_r, w2_r, b2_r, cw1_r, cb1_r, cw2_r, m_o, t8_o):
        i = pl.program_id(0)
        cd = cpr_r[...] - cpc_r[...]
        radial = jnp.sum(cd * cd, axis=1, keepdims=True)
        pre = hsr_r[...] + htr_r[...] + _bf(radial) * _bf(w1r_r[...]) + b1e_r[...]
        m1 = _silu(pre)
        m = _silu(_mm(m1, w2_r[...]) + b2_r[...])
        cm = _silu(_mm(m, cw1_r[...]) + cb1_r[...])
        s = jnp.sum(_bf(cm) * _bf(cw2_r[...]), axis=1, keepdims=True)
        t8 = cd * s
        rowid = i * BE + lax.broadcasted_iota(jnp.int32, (BE, 1), 0)
        valid = rowid < E
        colid = lax.broadcasted_iota(jnp.int32, (1, CP), 1)
        t8 = jnp.where(valid & (colid == 3), 1.0, t8)
        m_o[...] = jnp.where(valid, m, 0.0)
        t8_o[...] = t8

    full = lambda a: pl.BlockSpec(a.shape, lambda i: (0,) * a.ndim)
    return pl.pallas_call(
        body,
        grid=(E_PAD // BE,),
        in_specs=[
            pl.BlockSpec((BE, H), lambda i: (i, 0)),
            pl.BlockSpec((BE, H), lambda i: (i, 0)),
            pl.BlockSpec((BE, CP), lambda i: (i, 0)),
            pl.BlockSpec((BE, CP), lambda i: (i, 0)),
            full(w1r), full(b1e), full(w2), full(b2),
            full(cw1), full(cb1), full(cw2r),
        ],
        out_specs=[
            pl.BlockSpec((BE, H), lambda i: (i, 0)),
            pl.BlockSpec((BE, CP), lambda i: (i, 0)),
        ],
        out_shape=[
            jax.ShapeDtypeStruct((E_PAD, H), _f32),
            jax.ShapeDtypeStruct((E_PAD, CP), _f32),
        ],
    )(hsr, htr, cpr, cpc, w1r, b1e, w2, b2, cw1, cb1, cw2r)


def _tc_node(h64, maggp, aggxp, cp, nw1h, nw1m, nb1, nw2, nb2, wa, wb):
    def body(h_r, mg0_r, mg1_r, ax0_r, ax1_r, cp_r,
             nw1h_r, nw1m_r, nb1_r, nw2_r, nb2_r, wa_r, wb_r,
             h_o, cp_o, hs_o, ht_o):
        magg = mg0_r[0] + mg1_r[0]
        aggx = ax0_r[0] + ax1_r[0]
        cnt = aggx[:, 3:4]
        upd = aggx / jnp.maximum(cnt, 1.0)
        colid = lax.broadcasted_iota(jnp.int32, (1, CP), 1)
        cp_o[...] = cp_r[...] + jnp.where(colid < 3, upd, 0.0)
        tmp = _silu(_mm(h_r[...], nw1h_r[...])
                          + _mm(magg, nw1m_r[...]) + nb1_r[...])
        hn = h_r[...] + _mm(tmp, nw2_r[...]) + nb2_r[...]
        h_o[...] = hn
        hs_o[...] = _mm(hn, wa_r[...])
        ht_o[...] = _mm(hn, wb_r[...])

    full = lambda a: pl.BlockSpec(a.shape, lambda i: (0,) * a.ndim)
    return pl.pallas_call(
        body,
        grid=(N // BN,),
        in_specs=[
            pl.BlockSpec((BN, H), lambda i: (i, 0)),
            pl.BlockSpec((1, BN, H), lambda i: (0, i, 0)),
            pl.BlockSpec((1, BN, H), lambda i: (1, i, 0)),
            pl.BlockSpec((1, BN, CP), lambda i: (0, i, 0)),
            pl.BlockSpec((1, BN, CP), lambda i: (1, i, 0)),
            pl.BlockSpec((BN, CP), lambda i: (i, 0)),
            full(nw1h), full(nw1m), full(nb1), full(nw2), full(nb2),
            full(wa), full(wb),
        ],
        out_specs=[
            pl.BlockSpec((BN, H), lambda i: (i, 0)),
            pl.BlockSpec((BN, CP), lambda i: (i, 0)),
            pl.BlockSpec((BN, H), lambda i: (i, 0)),
            pl.BlockSpec((BN, H), lambda i: (i, 0)),
        ],
        out_shape=[
            jax.ShapeDtypeStruct((N, H), _f32),
            jax.ShapeDtypeStruct((N, CP), _f32),
            jax.ShapeDtypeStruct((N, H), _f32),
            jax.ShapeDtypeStruct((N, H), _f32),
        ],
    )(h64, maggp, maggp, aggxp, aggxp, cp,
      nw1h, nw1m, nb1, nw2, nb2, wa, wb)


def _tc_node_final(h64, maggp, aggxp, nw1h, nw1m, nb1, nw2, nb2,
                   ew, eb, mw1, mb1, mw2, mb2):
    nout = mw2.shape[1]

    def body(h_r, mg0_r, mg1_r,
             nw1h_r, nw1m_r, nb1_r, nw2_r, nb2_r,
             ew_r, eb_r, mw1_r, mb1_r, mw2_r, mb2_r, y_o):
        magg = mg0_r[0] + mg1_r[0]
        tmp = _silu(_mm(h_r[...], nw1h_r[...])
                          + _mm(magg, nw1m_r[...]) + nb1_r[...])
        hn = h_r[...] + _mm(tmp, nw2_r[...]) + nb2_r[...]
        t = _mm(hn, ew_r[...]) + eb_r[...]
        t2 = jnp.maximum(_mm(t, mw1_r[...]) + mb1_r[...], 0.0)
        y_o[...] = _mm(t2, mw2_r[...]) + mb2_r[...]

    full = lambda a: pl.BlockSpec(a.shape, lambda i: (0,) * a.ndim)
    return pl.pallas_call(
        body,
        grid=(N // BN,),
        in_specs=[
            pl.BlockSpec((BN, H), lambda i: (i, 0)),
            pl.BlockSpec((1, BN, H), lambda i: (0, i, 0)),
            pl.BlockSpec((1, BN, H), lambda i: (1, i, 0)),
            full(nw1h), full(nw1m), full(nb1), full(nw2), full(nb2),
            full(ew), full(eb), full(mw1), full(mb1), full(mw2), full(mb2),
        ],
        out_specs=pl.BlockSpec((BN, nout), lambda i: (i, 0)),
        out_shape=jax.ShapeDtypeStruct((N, nout), _f32),
    )(h64, maggp, maggp, nw1h, nw1m, nb1, nw2, nb2,
      ew, eb, mw1, mb1, mw2, mb2)


def _tc_init(h_in, ew, eb, wa, wb):
    def body(h_r, ew_r, eb_r, wa_r, wb_r, h_o, hs_o, ht_o):
        h64 = _mm(h_r[...], ew_r[...]) + eb_r[...]
        h_o[...] = h64
        hs_o[...] = _mm(h64, wa_r[...])
        ht_o[...] = _mm(h64, wb_r[...])

    full = lambda a: pl.BlockSpec(a.shape, lambda i: (0,) * a.ndim)
    din = h_in.shape[1]
    return pl.pallas_call(
        body,
        grid=(N // BN,),
        in_specs=[
            pl.BlockSpec((BN, din), lambda i: (i, 0)),
            full(ew), full(eb), full(wa), full(wb),
        ],
        out_specs=[
            pl.BlockSpec((BN, H), lambda i: (i, 0)),
            pl.BlockSpec((BN, H), lambda i: (i, 0)),
            pl.BlockSpec((BN, H), lambda i: (i, 0)),
        ],
        out_shape=[
            jax.ShapeDtypeStruct((N, H), _f32),
            jax.ShapeDtypeStruct((N, H), _f32),
            jax.ShapeDtypeStruct((N, H), _f32),
        ],
    )(h_in, ew, eb, wa, wb)


def kernel(h, x, edge_index, params):
    L = len(params["layers"])

    row = edge_index[:, 0]
    col = edge_index[:, 1]
    padn = E_PAD - E
    row2d = jnp.concatenate([row, jnp.zeros((padn,), jnp.int32)]).reshape(WIN_TOT, W)
    col2d = jnp.concatenate([col, jnp.zeros((padn,), jnp.int32)]).reshape(WIN_TOT, W)
    cp = jnp.pad(x, ((0, 0), (0, CP - x.shape[1])))
    zm = jnp.zeros((ROWS_PER_TILE, H), _f32)
    zx = jnp.zeros((ROWS_PER_TILE, CP), _f32)

    def r2(v):
        return v.reshape(1, -1)

    lw = []
    for p in params["layers"]:
        w1 = p["edge_w1"]
        lw.append(dict(
            wa=w1[:H], wb=w1[H:2 * H], w1r=r2(w1[2 * H]),
            b1e=r2(w1[2 * H + 1] + p["edge_b1"]),
            w2=p["edge_w2"], b2=r2(p["edge_b2"]),
            cw1=p["coord_w1"], cb1=r2(p["coord_b1"]), cw2r=r2(p["coord_w2"][:, 0]),
            nw1h=p["node_w1"][:H], nw1m=p["node_w1"][H:], nb1=r2(p["node_b1"]),
            nw2=p["node_w2"], nb2=r2(p["node_b2"]),
        ))

    h64, hs, ht = _tc_init(h, params["emb_in_w"], r2(params["emb_in_b"]),
                           lw[0]["wa"], lw[0]["wb"])

    y = None
    for l in range(L):
        p = lw[l]
        hsr, htr, cpr, cpc = _sc_gather(hs, ht, cp, row2d, col2d)
        m, t8 = _tc_edge(hsr, htr, cpr, cpc, p["w1r"], p["b1e"],
                         p["w2"], p["b2"], p["cw1"], p["cb1"], p["cw2r"])
        maggp, aggxp = _sc_scatter(m, t8, row2d, zm, zx)
        if l + 1 < L:
            h64, cp, hs, ht = _tc_node(
                h64, maggp, aggxp, cp,
                p["nw1h"], p["nw1m"], p["nb1"], p["nw2"], p["nb2"],
                lw[l + 1]["wa"], lw[l + 1]["wb"])
        else:
            y = _tc_node_final(
                h64, maggp, aggxp,
                p["nw1h"], p["nw1m"], p["nb1"], p["nw2"], p["nb2"],
                params["emb_out_w"], r2(params["emb_out_b"]),
                params["mlp_w1"], r2(params["mlp_b1"]),
                params["mlp_w2"], r2(params["mlp_b2"]))
    return y
